# baseline (speedup 1.0000x reference)
import functools

import jax
import jax.numpy as jnp
from jax.experimental import pallas as pl
from jax.experimental.pallas import tpu as pltpu

_B = 4
_R = 4096
_D = 2048
_CHUNK = 1024
_NCHUNK = _R // _CHUNK


def _body(k_ref, x_ref, o_ref, norms_ref, thr_ref):
    p = pl.program_id(1)
    c = pl.program_id(2)

    @pl.when(p == 0)
    def _norms():
        x = x_ref[0]
        s = jnp.sum(x * x, axis=-1, keepdims=True)
        norms_ref[pl.ds(c * _CHUNK, _CHUNK), :] = jnp.sqrt(s)

    @pl.when(jnp.logical_and(p == 1, c == 0))
    def _threshold():
        b = pl.program_id(0)
        k = k_ref[b]
        w = norms_ref[...].reshape(_R // 128, 128)
        bits = jax.lax.bitcast_convert_type(w, jnp.int32)

        def step(_, carry):
            lo, hi = carry
            mid = lo + (hi - lo + 1) // 2
            cnt = jnp.sum((bits >= mid).astype(jnp.int32))
            big = cnt >= k
            return jnp.where(big, mid, lo), jnp.where(big, hi, mid - 1)

        lo, hi = jax.lax.fori_loop(
            0, 31, step, (jnp.int32(0), jnp.int32(0x7F800000))
        )
        thr_ref[0] = jax.lax.bitcast_convert_type(lo, jnp.float32)

    @pl.when(p == 1)
    def _mask():
        thr = thr_ref[0]
        nc = norms_ref[pl.ds(c * _CHUNK, _CHUNK), :]
        mask = (nc >= thr).astype(jnp.float32)
        o_ref[0] = x_ref[0] * mask


@jax.jit
def kernel(weight_params, logits, W_score, b_score):
    keep_fraction_logit = logits @ W_score + b_score
    keep_fraction = jax.nn.sigmoid(keep_fraction_logit)
    rows_to_keep = jnp.maximum((keep_fraction * _R).astype(jnp.int32), 1)
    rows_to_keep = jnp.squeeze(rows_to_keep, axis=-1)

    grid = (_B, 2, _NCHUNK)
    return pl.pallas_call(
        _body,
        grid=grid,
        in_specs=[
            pl.BlockSpec(memory_space=pltpu.SMEM),
            pl.BlockSpec((1, _CHUNK, _D), lambda b, p, c: (b, c, 0)),
        ],
        out_specs=pl.BlockSpec(
            (1, _CHUNK, _D), lambda b, p, c: (b, jnp.where(p == 1, c, 0), 0)
        ),
        out_shape=jax.ShapeDtypeStruct((_B, _R, _D), jnp.float32),
        scratch_shapes=[
            pltpu.VMEM((_R, 1), jnp.float32),
            pltpu.SMEM((1,), jnp.float32),
        ],
    )(rows_to_keep, weight_params)


# speedup vs baseline: 1.1436x; 1.1436x over previous
import jax
import jax.numpy as jnp
from jax.experimental import pallas as pl
from jax.experimental.pallas import tpu as pltpu

_B = 4
_R = 4096
_D = 2048
_CHUNK = 1024
_NCHUNK = _R // _CHUNK


def _in_copy(x_ref, buf, sem_in, b, c):
    return pltpu.make_async_copy(
        x_ref.at[b, pl.ds(c * _CHUNK, _CHUNK)],
        buf.at[pl.ds(c * _CHUNK, _CHUNK)],
        sem_in.at[c],
    )


def _out_copy(o_ref, buf, sem_out, b, c):
    return pltpu.make_async_copy(
        buf.at[pl.ds(c * _CHUNK, _CHUNK)],
        o_ref.at[b, pl.ds(c * _CHUNK, _CHUNK)],
        sem_out.at[c],
    )


def _body(k_ref, x_ref, o_ref, buf, norms_ref, sem_in, sem_out):
    b = pl.program_id(0)

    for c in range(_NCHUNK):
        @pl.when(b > 0)
        def _release():
            _out_copy(o_ref, buf, sem_out, b - 1, c).wait()

        _in_copy(x_ref, buf, sem_in, b, c).start()

    for c in range(_NCHUNK):
        _in_copy(x_ref, buf, sem_in, b, c).wait()
        x = buf[pl.ds(c * _CHUNK, _CHUNK), :]
        s = jnp.sum(x * x, axis=-1, keepdims=True)
        norms_ref[pl.ds(c * _CHUNK, _CHUNK), :] = jnp.sqrt(s)

    k = k_ref[b]
    bits = jax.lax.bitcast_convert_type(
        norms_ref[...].reshape(_R // 128, 128), jnp.int32
    )

    def step(_, carry):
        lo, hi = carry
        mid = lo + (hi - lo + 1) // 2
        cnt = jnp.sum((bits >= mid).astype(jnp.int32))
        big = cnt >= k
        return jnp.where(big, mid, lo), jnp.where(big, hi, mid - 1)

    lo, _ = jax.lax.fori_loop(0, 31, step, (jnp.int32(0), jnp.int32(0x7F800000)))
    thr = jax.lax.bitcast_convert_type(lo, jnp.float32)

    for c in range(_NCHUNK):
        rows = pl.ds(c * _CHUNK, _CHUNK)
        mask = (norms_ref[rows, :] >= thr).astype(jnp.float32)
        buf[rows, :] = buf[rows, :] * mask
        _out_copy(o_ref, buf, sem_out, b, c).start()

    @pl.when(b == _B - 1)
    def _drain():
        for c in range(_NCHUNK):
            _out_copy(o_ref, buf, sem_out, b, c).wait()


@jax.jit
def kernel(weight_params, logits, W_score, b_score):
    keep_fraction_logit = logits @ W_score + b_score
    keep_fraction = jax.nn.sigmoid(keep_fraction_logit)
    rows_to_keep = jnp.maximum((keep_fraction * _R).astype(jnp.int32), 1)
    rows_to_keep = jnp.squeeze(rows_to_keep, axis=-1)

    return pl.pallas_call(
        _body,
        grid=(_B,),
        in_specs=[
            pl.BlockSpec(memory_space=pltpu.SMEM),
            pl.BlockSpec(memory_space=pl.ANY),
        ],
        out_specs=pl.BlockSpec(memory_space=pl.ANY),
        out_shape=jax.ShapeDtypeStruct((_B, _R, _D), jnp.float32),
        scratch_shapes=[
            pltpu.VMEM((_R, _D), jnp.float32),
            pltpu.VMEM((_R, 1), jnp.float32),
            pltpu.SemaphoreType.DMA((_NCHUNK,)),
            pltpu.SemaphoreType.DMA((_NCHUNK,)),
        ],
    )(rows_to_keep, weight_params)


# speedup vs baseline: 1.1604x; 1.0147x over previous
import jax
import jax.numpy as jnp
from jax.experimental import pallas as pl
from jax.experimental.pallas import tpu as pltpu

_B = 4
_R = 4096
_D = 2048
_SUB = 256
_NSUB = _R // _SUB


def _in_copy(x_ref, buf, sem_in, b, s):
    return pltpu.make_async_copy(
        x_ref.at[b, pl.ds(s * _SUB, _SUB)],
        buf.at[pl.ds(s * _SUB, _SUB)],
        sem_in.at[s],
    )


def _out_copy(o_ref, buf, sem_out, b, s):
    return pltpu.make_async_copy(
        buf.at[pl.ds(s * _SUB, _SUB)],
        o_ref.at[b, pl.ds(s * _SUB, _SUB)],
        sem_out.at[s],
    )


def _body(k_ref, x_ref, o_ref, buf, norms_ref, sem_in, sem_out):
    b = pl.program_id(0)

    for s in range(_NSUB):
        @pl.when(b > 0)
        def _release():
            _out_copy(o_ref, buf, sem_out, b - 1, s).wait()

        _in_copy(x_ref, buf, sem_in, b, s).start()

    for s in range(_NSUB):
        _in_copy(x_ref, buf, sem_in, b, s).wait()
        x = buf[pl.ds(s * _SUB, _SUB), :]
        ssq = jnp.sum(x * x, axis=-1, keepdims=True)
        norms_ref[pl.ds(s * _SUB, _SUB), :] = jnp.sqrt(ssq)

    k = k_ref[b]
    bits = jax.lax.bitcast_convert_type(
        norms_ref[...].reshape(_R // 128, 128), jnp.int32
    )

    def step(_, carry):
        lo, hi = carry
        mid = lo + (hi - lo + 1) // 2
        cnt = jnp.sum((bits >= mid).astype(jnp.int32))
        big = cnt >= k
        return jnp.where(big, mid, lo), jnp.where(big, hi, mid - 1)

    lo, _ = jax.lax.fori_loop(0, 31, step, (jnp.int32(0), jnp.int32(0x7F800000)))
    thr = jax.lax.bitcast_convert_type(lo, jnp.float32)

    for s in range(_NSUB):
        rows = pl.ds(s * _SUB, _SUB)
        mask = (norms_ref[rows, :] >= thr).astype(jnp.float32)
        buf[rows, :] = buf[rows, :] * mask
        _out_copy(o_ref, buf, sem_out, b, s).start()

    @pl.when(b == _B - 1)
    def _drain():
        for s in range(_NSUB):
            _out_copy(o_ref, buf, sem_out, b, s).wait()


@jax.jit
def kernel(weight_params, logits, W_score, b_score):
    keep_fraction_logit = logits @ W_score + b_score
    keep_fraction = jax.nn.sigmoid(keep_fraction_logit)
    rows_to_keep = jnp.maximum((keep_fraction * _R).astype(jnp.int32), 1)
    rows_to_keep = jnp.squeeze(rows_to_keep, axis=-1)

    return pl.pallas_call(
        _body,
        grid=(_B,),
        in_specs=[
            pl.BlockSpec(memory_space=pltpu.SMEM),
            pl.BlockSpec(memory_space=pl.ANY),
        ],
        out_specs=pl.BlockSpec(memory_space=pl.ANY),
        out_shape=jax.ShapeDtypeStruct((_B, _R, _D), jnp.float32),
        scratch_shapes=[
            pltpu.VMEM((_R, _D), jnp.float32),
            pltpu.VMEM((_R, 1), jnp.float32),
            pltpu.SemaphoreType.DMA((_NSUB,)),
            pltpu.SemaphoreType.DMA((_NSUB,)),
        ],
    )(rows_to_keep, weight_params)


# speedup vs baseline: 1.1712x; 1.0093x over previous
import jax
import jax.numpy as jnp
from jax.experimental import pallas as pl
from jax.experimental.pallas import tpu as pltpu

_B = 4
_R = 4096
_D = 2048
_SUB = 256
_NSUB = _R // _SUB


def _in_copy(x_ref, buf, sem_in, b, s):
    return pltpu.make_async_copy(
        x_ref.at[b, pl.ds(s * _SUB, _SUB)],
        buf.at[pl.ds(s * _SUB, _SUB)],
        sem_in.at[s],
    )


def _out_copy(o_ref, buf, sem_out, b, s):
    return pltpu.make_async_copy(
        buf.at[pl.ds(s * _SUB, _SUB)],
        o_ref.at[b, pl.ds(s * _SUB, _SUB)],
        sem_out.at[s],
    )


def _body(k_ref, x_ref, o_ref, buf, norms_ref, sem_in, sem_out):
    b = pl.program_id(0)

    for s in range(_NSUB):
        @pl.when(b > 0)
        def _release():
            _out_copy(o_ref, buf, sem_out, b - 1, s).wait()

        _in_copy(x_ref, buf, sem_in, b, s).start(priority=s % 2)

    for s in range(_NSUB):
        _in_copy(x_ref, buf, sem_in, b, s).wait()
        x = buf[pl.ds(s * _SUB, _SUB), :]
        ssq = jnp.sum(x * x, axis=-1, keepdims=True)
        norms_ref[pl.ds(s * _SUB, _SUB), :] = jnp.sqrt(ssq)

    k = k_ref[b]
    bits = jax.lax.bitcast_convert_type(
        norms_ref[...].reshape(_R // 128, 128), jnp.int32
    )

    def step(_, carry):
        lo, hi = carry
        mid = lo + (hi - lo + 1) // 2
        cnt = jnp.sum((bits >= mid).astype(jnp.int32))
        big = cnt >= k
        return jnp.where(big, mid, lo), jnp.where(big, hi, mid - 1)

    lo, _ = jax.lax.fori_loop(0, 31, step, (jnp.int32(0), jnp.int32(0x7F800000)))
    thr = jax.lax.bitcast_convert_type(lo, jnp.float32)

    for s in range(_NSUB):
        rows = pl.ds(s * _SUB, _SUB)
        mask = (norms_ref[rows, :] >= thr).astype(jnp.float32)
        buf[rows, :] = buf[rows, :] * mask
        _out_copy(o_ref, buf, sem_out, b, s).start(priority=s % 2)

    @pl.when(b == _B - 1)
    def _drain():
        for s in range(_NSUB):
            _out_copy(o_ref, buf, sem_out, b, s).wait()


@jax.jit
def kernel(weight_params, logits, W_score, b_score):
    keep_fraction_logit = logits @ W_score + b_score
    keep_fraction = jax.nn.sigmoid(keep_fraction_logit)
    rows_to_keep = jnp.maximum((keep_fraction * _R).astype(jnp.int32), 1)
    rows_to_keep = jnp.squeeze(rows_to_keep, axis=-1)

    return pl.pallas_call(
        _body,
        grid=(_B,),
        in_specs=[
            pl.BlockSpec(memory_space=pltpu.SMEM),
            pl.BlockSpec(memory_space=pl.ANY),
        ],
        out_specs=pl.BlockSpec(memory_space=pl.ANY),
        out_shape=jax.ShapeDtypeStruct((_B, _R, _D), jnp.float32),
        scratch_shapes=[
            pltpu.VMEM((_R, _D), jnp.float32),
            pltpu.VMEM((_R, 1), jnp.float32),
            pltpu.SemaphoreType.DMA((_NSUB,)),
            pltpu.SemaphoreType.DMA((_NSUB,)),
        ],
    )(rows_to_keep, weight_params)
